# Initial kernel scaffold; baseline (speedup 1.0000x reference)
import functools

import jax
import jax.numpy as jnp
from jax import lax
from jax.experimental import pallas as pl
from jax.experimental.pallas import tpu as pltpu
from jax.experimental.pallas import tpu_sc as plsc

N = 10000
E = 320000
D = 128
NR = 3
HALF = 5000
RPAD = 5120
SLOTS = NR * RPAD
GSLOTS = NR * 2 * RPAD
CH = 80
NT = 16
DUMMY = HALF

_mesh = plsc.VectorSubcoreMesh(core_axis_name="c", subcore_axis_name="s")


@functools.partial(
    pl.kernel,
    out_type=(jax.ShapeDtypeStruct((2, E), jnp.int32),
              jax.ShapeDtypeStruct((2, GSLOTS, 16), jnp.float32)),
    mesh=_mesh,
    scratch_types=[
        pltpu.VMEM((CH,), jnp.int32),
        pltpu.VMEM((CH,), jnp.int32),
        pltpu.VMEM((CH,), jnp.int32),
        pltpu.VMEM((CH,), jnp.int32),
        pltpu.VMEM((CH,), jnp.int32),
        pltpu.VMEM((CH, 16), jnp.float32),
        pltpu.VMEM_SHARED((GSLOTS, 16), jnp.float32),
    ],
)
def _sc_prep(ei, et, ones_h, z16, keys, cnt, dstb, etb, k0b, k1b, gkb, onesb, csh):
    h = lax.axis_index("c")
    s = lax.axis_index("s")
    rpt = GSLOTS // NT
    pltpu.sync_copy(z16, csh.at[pl.ds(s * rpt, rpt)])
    pltpu.sync_copy(ones_h, onesb)
    plsc.subcore_barrier()
    wid = h * NT + s
    base = wid * (E // (2 * NT))
    nch = (E // (2 * NT)) // CH

    def body(ci, carry):
        off = base + ci * CH
        pltpu.sync_copy(ei.at[1, pl.ds(off, CH)], dstb)
        pltpu.sync_copy(et.at[pl.ds(off, CH)], etb)
        for j in range(CH // 16):
            sl = pl.ds(j * 16, 16)
            d = dstb[sl]
            t = etb[sl]
            m0 = d < HALF
            k0b[sl] = jnp.where(m0, t * RPAD + d, DUMMY)
            k1b[sl] = jnp.where(m0, DUMMY, t * RPAD + d - HALF)
            gkb[sl] = t * (2 * RPAD) + jnp.where(m0, d, d + (RPAD - HALF))
        pltpu.sync_copy(k0b, keys.at[0, pl.ds(off, CH)])
        pltpu.sync_copy(k1b, keys.at[1, pl.ds(off, CH)])
        pltpu.sync_copy(onesb, csh.at[gkb], add=True)
        return carry

    lax.fori_loop(0, nch, body, 0)
    plsc.subcore_barrier()
    pltpu.sync_copy(csh.at[pl.ds(s * rpt, rpt)], cnt.at[h, pl.ds(s * rpt, rpt)])


@functools.partial(
    pl.kernel,
    out_type=jax.ShapeDtypeStruct((NR, 2, RPAD, D), jnp.float32),
    mesh=_mesh,
    scratch_types=[
        pltpu.VMEM((CH,), jnp.int32),
        pltpu.VMEM((CH,), jnp.int32),
        pltpu.VMEM((CH, D), jnp.float32),
        pltpu.VMEM_SHARED((SLOTS, D), jnp.float32),
        pltpu.SemaphoreType.DMA,
    ],
)
def _sc_agg(x, srcs, keys, zrows, a_out, srcb, keyb, rows, acc, sem):
    h = lax.axis_index("c")
    s = lax.axis_index("s")
    zr = SLOTS // NT
    pltpu.sync_copy(zrows, acc.at[pl.ds(s * zr, zr)])
    plsc.subcore_barrier()
    per_tile = E // NT
    base = s * per_tile
    nch = per_tile // CH

    def body(ci, carry):
        off = base + ci * CH
        pltpu.sync_copy(srcs.at[pl.ds(off, CH)], srcb)
        pltpu.sync_copy(keys.at[h, pl.ds(off, CH)], keyb)
        pltpu.async_copy(x.at[srcb], rows, sem).wait()
        pltpu.sync_copy(rows, acc.at[keyb], add=True)
        return carry

    lax.fori_loop(0, nch, body, 0)
    plsc.subcore_barrier()
    opr = RPAD // NT
    for r in range(NR):
        pltpu.sync_copy(acc.at[pl.ds(r * RPAD + s * opr, opr)],
                        a_out.at[r, h, pl.ds(s * opr, opr)])


@functools.partial(
    pl.kernel,
    out_type=(jax.ShapeDtypeStruct((E, D), jnp.float32),
              jax.ShapeDtypeStruct((E, D), jnp.float32)),
    mesh=_mesh,
    scratch_types=[
        pltpu.VMEM((CH,), jnp.int32),
        pltpu.VMEM((CH,), jnp.int32),
        pltpu.VMEM((CH, D), jnp.float32),
        pltpu.VMEM((CH, D), jnp.float32),
        pltpu.SemaphoreType.DMA,
        pltpu.SemaphoreType.DMA,
    ],
)
def _sc_edge_gather(p, q, ei, gp, gq, srcb, dstb, rp, rq, semp, semq):
    h = lax.axis_index("c")
    s = lax.axis_index("s")
    wid = h * NT + s
    per_w = E // (2 * NT)
    base = wid * per_w
    nch = per_w // CH

    def body(ci, carry):
        off = base + ci * CH
        pltpu.sync_copy(ei.at[0, pl.ds(off, CH)], srcb)
        pltpu.sync_copy(ei.at[1, pl.ds(off, CH)], dstb)
        cp = pltpu.async_copy(p.at[srcb], rp, semp)
        cq = pltpu.async_copy(q.at[dstb], rq, semq)
        cp.wait()
        cq.wait()
        pltpu.sync_copy(rp, gp.at[pl.ds(off, CH)])
        pltpu.sync_copy(rq, gq.at[pl.ds(off, CH)])
        return carry

    lax.fori_loop(0, nch, body, 0)


_BR = 200


def _tc_combine_body(relu, x_ref, a_ref, c_ref, w_ref, rw_ref, b_ref, o_ref):
    acc = jnp.dot(x_ref[...], rw_ref[...], preferred_element_type=jnp.float32)
    acc = acc + b_ref[...]
    for r in range(NR):
        c = c_ref[0, r, 0, :, 0:1] + c_ref[1, r, 0, :, 0:1]
        a = a_ref[r, 0] / jnp.maximum(c, 1.0)
        acc = acc + jnp.dot(a, w_ref[r], preferred_element_type=jnp.float32)
    o_ref[...] = jnp.maximum(acc, 0.0) if relu else acc


def _tc_combine(x, a, cnt, w, rw, b, relu):
    nb = HALF // _BR
    return pl.pallas_call(
        functools.partial(_tc_combine_body, relu),
        grid=(2 * nb,),
        in_specs=[
            pl.BlockSpec((_BR, D), lambda g: (g, 0)),
            pl.BlockSpec((NR, 1, _BR, D), lambda g: (0, g // 25, g % 25, 0)),
            pl.BlockSpec((2, NR, 1, _BR, 16), lambda g: (0, 0, g // 25, g % 25, 0)),
            pl.BlockSpec((NR, D, D), lambda g: (0, 0, 0)),
            pl.BlockSpec((D, D), lambda g: (0, 0)),
            pl.BlockSpec((1, D), lambda g: (0, 0)),
        ],
        out_specs=pl.BlockSpec((_BR, D), lambda g: (g, 0)),
        out_shape=jax.ShapeDtypeStruct((N, D), jnp.float32),
    )(x, a, cnt, w, rw, b)


def _tc_pq_body(x_ref, wa_ref, wb_ref, b_ref, p_ref, q_ref):
    x = x_ref[...]
    p_ref[...] = jnp.dot(x, wa_ref[...], preferred_element_type=jnp.float32) + b_ref[...]
    q_ref[...] = jnp.dot(x, wb_ref[...], preferred_element_type=jnp.float32)


def _tc_pq(x, wa, wb, b):
    return pl.pallas_call(
        _tc_pq_body,
        grid=(N // _BR,),
        in_specs=[
            pl.BlockSpec((_BR, D), lambda g: (g, 0)),
            pl.BlockSpec((D, D), lambda g: (0, 0)),
            pl.BlockSpec((D, D), lambda g: (0, 0)),
            pl.BlockSpec((1, D), lambda g: (0, 0)),
        ],
        out_specs=[pl.BlockSpec((_BR, D), lambda g: (g, 0)),
                   pl.BlockSpec((_BR, D), lambda g: (g, 0))],
        out_shape=[jax.ShapeDtypeStruct((N, D), jnp.float32),
                   jax.ShapeDtypeStruct((N, D), jnp.float32)],
    )(x, wa, wb, b)


_BE = 500


def _tc_edge_body(gp_ref, gq_ref, w_ref, b_ref, o_ref):
    eh = jnp.maximum(gp_ref[...] + gq_ref[...], 0.0)
    o_ref[...] = jnp.dot(eh, w_ref[...], preferred_element_type=jnp.float32) + b_ref[...]


def _tc_edge(gp, gq, w, b):
    return pl.pallas_call(
        _tc_edge_body,
        grid=(E // _BE,),
        in_specs=[
            pl.BlockSpec((_BE, D), lambda g: (g, 0)),
            pl.BlockSpec((_BE, D), lambda g: (g, 0)),
            pl.BlockSpec((D, 8), lambda g: (0, 0)),
            pl.BlockSpec((1, 8), lambda g: (0, 0)),
        ],
        out_specs=pl.BlockSpec((_BE, 8), lambda g: (g, 0)),
        out_shape=jax.ShapeDtypeStruct((E, 8), jnp.float32),
    )(gp, gq, w, b)


def _tc_node_body(x_ref, w1_ref, b1_ref, w2_ref, b2_ref, o_ref):
    hdd = jnp.dot(x_ref[...], w1_ref[...], preferred_element_type=jnp.float32)
    hdd = jnp.maximum(hdd + b1_ref[...], 0.0)
    o_ref[...] = jnp.dot(hdd, w2_ref[...], preferred_element_type=jnp.float32) + b2_ref[...]


def _tc_node(x, w1, b1, w2, b2):
    return pl.pallas_call(
        _tc_node_body,
        grid=(N // _BR,),
        in_specs=[
            pl.BlockSpec((_BR, D), lambda g: (g, 0)),
            pl.BlockSpec((D, 64), lambda g: (0, 0)),
            pl.BlockSpec((1, 64), lambda g: (0, 0)),
            pl.BlockSpec((64, 8), lambda g: (0, 0)),
            pl.BlockSpec((1, 8), lambda g: (0, 0)),
        ],
        out_specs=pl.BlockSpec((_BR, 8), lambda g: (g, 0)),
        out_shape=jax.ShapeDtypeStruct((N, 8), jnp.float32),
    )(x, w1, b1, w2, b2)


def kernel(x, edge_index, edge_type, W1, R1, b1, W2, R2, b2, W3, R3, b3,
           We1, be1, We2, be2, Wn1, bn1, Wn2, bn2):
    src = edge_index[0]
    zrows = jnp.zeros((SLOTS // NT, D), jnp.float32)
    z16 = jnp.zeros((GSLOTS // NT, 16), jnp.float32)
    ones_h = jnp.ones((CH, 16), jnp.float32)

    keys, cnt = _sc_prep(edge_index, edge_type, ones_h, z16)
    cnt5 = cnt.reshape(2, NR, 2, RPAD, 16)

    xcur = x
    for (W, Rw, b, relu) in ((W1, R1, b1, True), (W2, R2, b2, True),
                             (W3, R3, b3, False)):
        a = _sc_agg(xcur, src, keys, zrows)
        xcur = _tc_combine(xcur, a, cnt5, W, Rw, b.reshape(1, D), relu)

    p, q = _tc_pq(xcur, We1[:D], We1[D:], be1.reshape(1, D))
    gp, gq = _sc_edge_gather(p, q, edge_index)

    we2p = jnp.zeros((D, 8), jnp.float32).at[:, :3].set(We2)
    be2p = jnp.zeros((1, 8), jnp.float32).at[0, :3].set(be2)
    edge_out = _tc_edge(gp, gq, we2p, be2p)[:, :3]

    wn2p = jnp.zeros((64, 8), jnp.float32).at[:, :2].set(Wn2)
    bn2p = jnp.zeros((1, 8), jnp.float32).at[0, :2].set(bn2)
    node_out = _tc_node(xcur, Wn1, bn1.reshape(1, 64), wn2p, bn2p)[:, :2]

    return edge_out, node_out


# speedup vs baseline: 3.4610x; 3.4610x over previous
import functools

import jax
import jax.numpy as jnp
from jax import lax
from jax.experimental import pallas as pl
from jax.experimental.pallas import tpu as pltpu
from jax.experimental.pallas import tpu_sc as plsc

N = 10000
E = 320000
D = 128
NR = 3
HALF = 5000
RPAD = 5120
GSLOTS = NR * 2 * RPAD
AROWS = 15008
ADUMMY = 15000
ASLAB = 944
CH = 80
NT = 16

_mesh = plsc.VectorSubcoreMesh(core_axis_name="c", subcore_axis_name="s")


@functools.partial(
    pl.kernel,
    out_type=(jax.ShapeDtypeStruct((2 * E,), jnp.int32),
              jax.ShapeDtypeStruct((2, AROWS, D), jnp.float32)),
    mesh=_mesh,
    scratch_types=[
        pltpu.VMEM((CH,), jnp.int32),
        pltpu.VMEM((CH,), jnp.int32),
        pltpu.VMEM((CH,), jnp.int32),
        pltpu.VMEM((CH, D), jnp.float32),
        pltpu.VMEM_SHARED((AROWS, D), jnp.float32),
    ],
)
def _sc_prep(dsts, et, ones_h, zrows, keys, cnt, dstb, etb, kb, onesb, acc):
    h = lax.axis_index("c")
    s = lax.axis_index("s")

    @pl.when(s < NT - 1)
    def _():
        pltpu.sync_copy(zrows, acc.at[pl.ds(s * ASLAB, ASLAB)])

    @pl.when(s == NT - 1)
    def _():
        pltpu.sync_copy(zrows.at[pl.ds(0, AROWS - (NT - 1) * ASLAB)],
                        acc.at[pl.ds(s * ASLAB, AROWS - (NT - 1) * ASLAB)])

    pltpu.sync_copy(ones_h, onesb)
    plsc.subcore_barrier()
    per_tile = E // NT
    base = s * per_tile
    nch = per_tile // CH
    lo = h * HALF

    def body(ci, carry):
        off = base + ci * CH
        pltpu.sync_copy(dsts.at[pl.ds(off, CH)], dstb)
        pltpu.sync_copy(et.at[pl.ds(off, CH)], etb)
        for j in range(CH // 16):
            sl = pl.ds(j * 16, 16)
            d = dstb[sl]
            t = etb[sl]
            mine = (d >= lo) & (d < lo + HALF)
            kb[sl] = jnp.where(mine, t * HALF + d - lo, ADUMMY)
        pltpu.sync_copy(kb, keys.at[pl.ds(h * E + off, CH)])
        pltpu.sync_copy(onesb, acc.at[kb], add=True)
        return carry

    lax.fori_loop(0, nch, body, 0)
    plsc.subcore_barrier()

    @pl.when(s < NT - 1)
    def _():
        pltpu.sync_copy(acc.at[pl.ds(s * ASLAB, ASLAB)],
                        cnt.at[h, pl.ds(s * ASLAB, ASLAB)])

    @pl.when(s == NT - 1)
    def _():
        tail = AROWS - (NT - 1) * ASLAB
        pltpu.sync_copy(acc.at[pl.ds(s * ASLAB, tail)],
                        cnt.at[h, pl.ds(s * ASLAB, tail)])


@functools.partial(
    pl.kernel,
    out_type=jax.ShapeDtypeStruct((2, AROWS, D), jnp.float32),
    mesh=_mesh,
    scratch_types=[
        pltpu.VMEM((CH,), jnp.int32),
        pltpu.VMEM((CH,), jnp.int32),
        pltpu.VMEM((CH, D), jnp.float32),
        pltpu.VMEM_SHARED((AROWS, D), jnp.float32),
        pltpu.SemaphoreType.DMA,
    ],
)
def _sc_agg(x, srcs, keys, zrows, a_out, srcb, keyb, rows, acc, sem):
    h = lax.axis_index("c")
    s = lax.axis_index("s")

    @pl.when(s < NT - 1)
    def _():
        pltpu.sync_copy(zrows, acc.at[pl.ds(s * ASLAB, ASLAB)])

    @pl.when(s == NT - 1)
    def _():
        pltpu.sync_copy(zrows.at[pl.ds(0, AROWS - (NT - 1) * ASLAB)],
                        acc.at[pl.ds(s * ASLAB, AROWS - (NT - 1) * ASLAB)])

    plsc.subcore_barrier()
    per_tile = E // NT
    base = s * per_tile
    nch = per_tile // CH

    def body(ci, carry):
        off = base + ci * CH
        pltpu.sync_copy(srcs.at[pl.ds(off, CH)], srcb)
        pltpu.sync_copy(keys.at[pl.ds(h * E + off, CH)], keyb)
        pltpu.async_copy(x.at[srcb], rows, sem).wait()
        pltpu.sync_copy(rows, acc.at[keyb], add=True)
        return carry

    lax.fori_loop(0, nch, body, 0)
    plsc.subcore_barrier()

    @pl.when(s < NT - 1)
    def _():
        pltpu.sync_copy(acc.at[pl.ds(s * ASLAB, ASLAB)],
                        a_out.at[h, pl.ds(s * ASLAB, ASLAB)])

    @pl.when(s == NT - 1)
    def _():
        tail = AROWS - (NT - 1) * ASLAB
        pltpu.sync_copy(acc.at[pl.ds(s * ASLAB, tail)],
                        a_out.at[h, pl.ds(s * ASLAB, tail)])


@functools.partial(
    pl.kernel,
    out_type=(jax.ShapeDtypeStruct((E, D), jnp.float32),
              jax.ShapeDtypeStruct((E, D), jnp.float32)),
    mesh=_mesh,
    scratch_types=[
        pltpu.VMEM((CH,), jnp.int32),
        pltpu.VMEM((CH,), jnp.int32),
        pltpu.VMEM((CH, D), jnp.float32),
        pltpu.VMEM((CH, D), jnp.float32),
        pltpu.SemaphoreType.DMA,
        pltpu.SemaphoreType.DMA,
    ],
)
def _sc_edge_gather(p, q, srcs, dsts, gp, gq, srcb, dstb, rp, rq, semp, semq):
    h = lax.axis_index("c")
    s = lax.axis_index("s")
    wid = h * NT + s
    per_w = E // (2 * NT)
    base = wid * per_w
    nch = per_w // CH

    def body(ci, carry):
        off = base + ci * CH
        pltpu.sync_copy(srcs.at[pl.ds(off, CH)], srcb)
        pltpu.sync_copy(dsts.at[pl.ds(off, CH)], dstb)
        cp = pltpu.async_copy(p.at[srcb], rp, semp)
        cq = pltpu.async_copy(q.at[dstb], rq, semq)
        cp.wait()
        cq.wait()
        pltpu.sync_copy(rp, gp.at[pl.ds(off, CH)])
        pltpu.sync_copy(rq, gq.at[pl.ds(off, CH)])
        return carry

    lax.fori_loop(0, nch, body, 0)


_BR = 200


def _tc_combine_body(relu, x_ref, a0_ref, a1_ref, a2_ref, c0_ref, c1_ref,
                     c2_ref, w_ref, rw_ref, b_ref, o_ref):
    acc = jnp.dot(x_ref[...], rw_ref[...], preferred_element_type=jnp.float32)
    acc = acc + b_ref[...]
    for r, (ar, cr) in enumerate(zip((a0_ref, a1_ref, a2_ref),
                                     (c0_ref, c1_ref, c2_ref))):
        c = cr[0][:, 0:1]
        a = ar[0] / jnp.maximum(c, 1.0)
        acc = acc + jnp.dot(a, w_ref[r], preferred_element_type=jnp.float32)
    o_ref[...] = jnp.maximum(acc, 0.0) if relu else acc


def _tc_combine(x, a, cnt, w, rw, b, relu):
    nb = HALF // _BR
    a_spec = lambda r: pl.BlockSpec(
        (1, _BR, D), lambda g, r=r: (g // nb, r * nb + (g % nb), 0))
    return pl.pallas_call(
        functools.partial(_tc_combine_body, relu),
        grid=(2 * nb,),
        in_specs=[
            pl.BlockSpec((_BR, D), lambda g: (g, 0)),
            a_spec(0), a_spec(1), a_spec(2),
            a_spec(0), a_spec(1), a_spec(2),
            pl.BlockSpec((NR, D, D), lambda g: (0, 0, 0)),
            pl.BlockSpec((D, D), lambda g: (0, 0)),
            pl.BlockSpec((1, D), lambda g: (0, 0)),
        ],
        out_specs=pl.BlockSpec((_BR, D), lambda g: (g, 0)),
        out_shape=jax.ShapeDtypeStruct((N, D), jnp.float32),
    )(x, a, a, a, cnt, cnt, cnt, w, rw, b)


def _tc_pq_body(x_ref, wa_ref, wb_ref, b_ref, p_ref, q_ref):
    x = x_ref[...]
    p_ref[...] = jnp.dot(x, wa_ref[...], preferred_element_type=jnp.float32) + b_ref[...]
    q_ref[...] = jnp.dot(x, wb_ref[...], preferred_element_type=jnp.float32)


def _tc_pq(x, wa, wb, b):
    return pl.pallas_call(
        _tc_pq_body,
        grid=(N // _BR,),
        in_specs=[
            pl.BlockSpec((_BR, D), lambda g: (g, 0)),
            pl.BlockSpec((D, D), lambda g: (0, 0)),
            pl.BlockSpec((D, D), lambda g: (0, 0)),
            pl.BlockSpec((1, D), lambda g: (0, 0)),
        ],
        out_specs=[pl.BlockSpec((_BR, D), lambda g: (g, 0)),
                   pl.BlockSpec((_BR, D), lambda g: (g, 0))],
        out_shape=[jax.ShapeDtypeStruct((N, D), jnp.float32),
                   jax.ShapeDtypeStruct((N, D), jnp.float32)],
    )(x, wa, wb, b)


_BE = 512


def _tc_edge_body(gp_ref, gq_ref, w_ref, b_ref, o_ref):
    eh = jnp.maximum(gp_ref[...] + gq_ref[...], 0.0)
    o_ref[...] = jnp.dot(eh, w_ref[...], preferred_element_type=jnp.float32) + b_ref[...]


def _tc_edge(gp, gq, w, b):
    return pl.pallas_call(
        _tc_edge_body,
        grid=(E // _BE,),
        in_specs=[
            pl.BlockSpec((_BE, D), lambda g: (g, 0)),
            pl.BlockSpec((_BE, D), lambda g: (g, 0)),
            pl.BlockSpec((D, 8), lambda g: (0, 0)),
            pl.BlockSpec((1, 8), lambda g: (0, 0)),
        ],
        out_specs=pl.BlockSpec((_BE, 8), lambda g: (g, 0)),
        out_shape=jax.ShapeDtypeStruct((E, 8), jnp.float32),
    )(gp, gq, w, b)


def _tc_node_body(x_ref, w1_ref, b1_ref, w2_ref, b2_ref, o_ref):
    hdd = jnp.dot(x_ref[...], w1_ref[...], preferred_element_type=jnp.float32)
    hdd = jnp.maximum(hdd + b1_ref[...], 0.0)
    o_ref[...] = jnp.dot(hdd, w2_ref[...], preferred_element_type=jnp.float32) + b2_ref[...]


def _tc_node(x, w1, b1, w2, b2):
    return pl.pallas_call(
        _tc_node_body,
        grid=(N // _BR,),
        in_specs=[
            pl.BlockSpec((_BR, D), lambda g: (g, 0)),
            pl.BlockSpec((D, 64), lambda g: (0, 0)),
            pl.BlockSpec((1, 64), lambda g: (0, 0)),
            pl.BlockSpec((64, 8), lambda g: (0, 0)),
            pl.BlockSpec((1, 8), lambda g: (0, 0)),
        ],
        out_specs=pl.BlockSpec((_BR, 8), lambda g: (g, 0)),
        out_shape=jax.ShapeDtypeStruct((N, 8), jnp.float32),
    )(x, w1, b1, w2, b2)


def kernel(x, edge_index, edge_type, W1, R1, b1, W2, R2, b2, W3, R3, b3,
           We1, be1, We2, be2, Wn1, bn1, Wn2, bn2):
    src = edge_index[0]
    dst = edge_index[1]
    zrows = jnp.zeros((ASLAB, D), jnp.float32)
    ones_h = jnp.ones((CH, D), jnp.float32)

    keys, cnt = _sc_prep(dst, edge_type, ones_h, zrows)

    xcur = x
    for (W, Rw, b, relu) in ((W1, R1, b1, True), (W2, R2, b2, True),
                             (W3, R3, b3, False)):
        a = _sc_agg(xcur, src, keys, zrows)
        xcur = _tc_combine(xcur, a, cnt, W, Rw, b.reshape(1, D), relu)

    p, q = _tc_pq(xcur, We1[:D], We1[D:], be1.reshape(1, D))
    gp, gq = _sc_edge_gather(p, q, src, dst)

    we2p = jnp.zeros((D, 8), jnp.float32).at[:, :3].set(We2)
    be2p = jnp.zeros((1, 8), jnp.float32).at[0, :3].set(be2)
    edge_out = _tc_edge(gp, gq, we2p, be2p)[:, :3]

    wn2p = jnp.zeros((64, 8), jnp.float32).at[:, :2].set(Wn2)
    bn2p = jnp.zeros((1, 8), jnp.float32).at[0, :2].set(bn2)
    node_out = _tc_node(xcur, Wn1, bn1.reshape(1, 64), wn2p, bn2p)[:, :2]

    return edge_out, node_out


# speedup vs baseline: 3.5842x; 1.0356x over previous
import functools

import jax
import jax.numpy as jnp
from jax import lax
from jax.experimental import pallas as pl
from jax.experimental.pallas import tpu as pltpu
from jax.experimental.pallas import tpu_sc as plsc

N = 10000
E = 320000
D = 128
NR = 3
HALF = 5000
RPAD = 5120
GSLOTS = NR * 2 * RPAD
AROWS = 15008
ADUMMY = 15000
ASLAB = 944
CH = 80
CHA = 40
NT = 16

_mesh = plsc.VectorSubcoreMesh(core_axis_name="c", subcore_axis_name="s")


@functools.partial(
    pl.kernel,
    out_type=(jax.ShapeDtypeStruct((2 * E,), jnp.int32),
              jax.ShapeDtypeStruct((2, AROWS, D), jnp.float32)),
    mesh=_mesh,
    scratch_types=[
        pltpu.VMEM((CH,), jnp.int32),
        pltpu.VMEM((CH,), jnp.int32),
        pltpu.VMEM((CH,), jnp.int32),
        pltpu.VMEM((CH, D), jnp.float32),
        pltpu.VMEM_SHARED((AROWS, D), jnp.float32),
    ],
)
def _sc_prep(dsts, et, ones_h, zrows, keys, cnt, dstb, etb, kb, onesb, acc):
    h = lax.axis_index("c")
    s = lax.axis_index("s")

    @pl.when(s < NT - 1)
    def _():
        pltpu.sync_copy(zrows, acc.at[pl.ds(s * ASLAB, ASLAB)])

    @pl.when(s == NT - 1)
    def _():
        pltpu.sync_copy(zrows.at[pl.ds(0, AROWS - (NT - 1) * ASLAB)],
                        acc.at[pl.ds(s * ASLAB, AROWS - (NT - 1) * ASLAB)])

    pltpu.sync_copy(ones_h, onesb)
    plsc.subcore_barrier()
    per_tile = E // NT
    base = s * per_tile
    nch = per_tile // CH
    lo = h * HALF

    def body(ci, carry):
        off = base + ci * CH
        pltpu.sync_copy(dsts.at[pl.ds(off, CH)], dstb)
        pltpu.sync_copy(et.at[pl.ds(off, CH)], etb)
        for j in range(CH // 16):
            sl = pl.ds(j * 16, 16)
            d = dstb[sl]
            t = etb[sl]
            mine = (d >= lo) & (d < lo + HALF)
            kb[sl] = jnp.where(mine, t * HALF + d - lo, ADUMMY)
        pltpu.sync_copy(kb, keys.at[pl.ds(h * E + off, CH)])
        pltpu.sync_copy(onesb, acc.at[kb], add=True)
        return carry

    lax.fori_loop(0, nch, body, 0)
    plsc.subcore_barrier()

    @pl.when(s < NT - 1)
    def _():
        pltpu.sync_copy(acc.at[pl.ds(s * ASLAB, ASLAB)],
                        cnt.at[h, pl.ds(s * ASLAB, ASLAB)])

    @pl.when(s == NT - 1)
    def _():
        tail = AROWS - (NT - 1) * ASLAB
        pltpu.sync_copy(acc.at[pl.ds(s * ASLAB, tail)],
                        cnt.at[h, pl.ds(s * ASLAB, tail)])


@functools.partial(
    pl.kernel,
    out_type=jax.ShapeDtypeStruct((2, AROWS, D), jnp.float32),
    mesh=_mesh,
    scratch_types=[
        pltpu.VMEM((CHA,), jnp.int32),
        pltpu.VMEM((CHA,), jnp.int32),
        pltpu.VMEM((CHA,), jnp.int32),
        pltpu.VMEM((CHA,), jnp.int32),
        pltpu.VMEM((CHA, D), jnp.float32),
        pltpu.VMEM((CHA, D), jnp.float32),
        pltpu.VMEM_SHARED((AROWS, D), jnp.float32),
        pltpu.SemaphoreType.DMA,
        pltpu.SemaphoreType.DMA,
    ],
)
def _sc_agg(x, srcs, keys, zrows, a_out, srcb1, srcb2, keyb1, keyb2, rows1,
            rows2, acc, sem1, sem2):
    h = lax.axis_index("c")
    s = lax.axis_index("s")

    @pl.when(s < NT - 1)
    def _():
        pltpu.sync_copy(zrows, acc.at[pl.ds(s * ASLAB, ASLAB)])

    @pl.when(s == NT - 1)
    def _():
        pltpu.sync_copy(zrows.at[pl.ds(0, AROWS - (NT - 1) * ASLAB)],
                        acc.at[pl.ds(s * ASLAB, AROWS - (NT - 1) * ASLAB)])

    plsc.subcore_barrier()
    per_tile = E // NT
    base = s * per_tile
    nch = per_tile // CHA

    def load(off, srcb, keyb, rows, sem):
        pltpu.sync_copy(srcs.at[pl.ds(off, CHA)], srcb)
        pltpu.sync_copy(keys.at[pl.ds(h * E + off, CHA)], keyb)
        return pltpu.async_copy(x.at[srcb], rows, sem)

    load(base, srcb1, keyb1, rows1, sem1)

    def body(k, carry):
        offb = base + (2 * k + 1) * CHA
        offa = base + (2 * k + 2) * CHA
        load(offb, srcb2, keyb2, rows2, sem2)
        pltpu.make_async_copy(x.at[srcb1], rows1, sem1).wait()
        pltpu.sync_copy(rows1, acc.at[keyb1], add=True)
        load(offa, srcb1, keyb1, rows1, sem1)
        pltpu.make_async_copy(x.at[srcb2], rows2, sem2).wait()
        pltpu.sync_copy(rows2, acc.at[keyb2], add=True)
        return carry

    lax.fori_loop(0, (nch - 2) // 2, body, 0)
    load(base + (nch - 1) * CHA, srcb2, keyb2, rows2, sem2)
    pltpu.make_async_copy(x.at[srcb1], rows1, sem1).wait()
    pltpu.sync_copy(rows1, acc.at[keyb1], add=True)
    pltpu.make_async_copy(x.at[srcb2], rows2, sem2).wait()
    pltpu.sync_copy(rows2, acc.at[keyb2], add=True)
    plsc.subcore_barrier()

    @pl.when(s < NT - 1)
    def _():
        pltpu.sync_copy(acc.at[pl.ds(s * ASLAB, ASLAB)],
                        a_out.at[h, pl.ds(s * ASLAB, ASLAB)])

    @pl.when(s == NT - 1)
    def _():
        tail = AROWS - (NT - 1) * ASLAB
        pltpu.sync_copy(acc.at[pl.ds(s * ASLAB, tail)],
                        a_out.at[h, pl.ds(s * ASLAB, tail)])


@functools.partial(
    pl.kernel,
    out_type=(jax.ShapeDtypeStruct((E, D), jnp.float32),
              jax.ShapeDtypeStruct((E, D), jnp.float32)),
    mesh=_mesh,
    scratch_types=[
        pltpu.VMEM((CH,), jnp.int32),
        pltpu.VMEM((CH,), jnp.int32),
        pltpu.VMEM((CH, D), jnp.float32),
        pltpu.VMEM((CH, D), jnp.float32),
        pltpu.SemaphoreType.DMA,
        pltpu.SemaphoreType.DMA,
    ],
)
def _sc_edge_gather(p, q, srcs, dsts, gp, gq, srcb, dstb, rp, rq, semp, semq):
    h = lax.axis_index("c")
    s = lax.axis_index("s")
    wid = h * NT + s
    per_w = E // (2 * NT)
    base = wid * per_w
    nch = per_w // CH

    def body(ci, carry):
        off = base + ci * CH
        pltpu.sync_copy(srcs.at[pl.ds(off, CH)], srcb)
        pltpu.sync_copy(dsts.at[pl.ds(off, CH)], dstb)
        cp = pltpu.async_copy(p.at[srcb], rp, semp)
        cq = pltpu.async_copy(q.at[dstb], rq, semq)
        cp.wait()
        cq.wait()
        pltpu.sync_copy(rp, gp.at[pl.ds(off, CH)])
        pltpu.sync_copy(rq, gq.at[pl.ds(off, CH)])
        return carry

    lax.fori_loop(0, nch, body, 0)


_BR = 200


def _tc_combine_body(relu, x_ref, a0_ref, a1_ref, a2_ref, c0_ref, c1_ref,
                     c2_ref, w_ref, rw_ref, b_ref, o_ref):
    acc = jnp.dot(x_ref[...], rw_ref[...], preferred_element_type=jnp.float32)
    acc = acc + b_ref[...]
    for r, (ar, cr) in enumerate(zip((a0_ref, a1_ref, a2_ref),
                                     (c0_ref, c1_ref, c2_ref))):
        c = cr[0][:, 0:1]
        a = ar[0] / jnp.maximum(c, 1.0)
        acc = acc + jnp.dot(a, w_ref[r], preferred_element_type=jnp.float32)
    o_ref[...] = jnp.maximum(acc, 0.0) if relu else acc


def _tc_combine(x, a, cnt, w, rw, b, relu):
    nb = HALF // _BR
    a_spec = lambda r: pl.BlockSpec(
        (1, _BR, D), lambda g, r=r: (g // nb, r * nb + (g % nb), 0))
    return pl.pallas_call(
        functools.partial(_tc_combine_body, relu),
        grid=(2 * nb,),
        in_specs=[
            pl.BlockSpec((_BR, D), lambda g: (g, 0)),
            a_spec(0), a_spec(1), a_spec(2),
            a_spec(0), a_spec(1), a_spec(2),
            pl.BlockSpec((NR, D, D), lambda g: (0, 0, 0)),
            pl.BlockSpec((D, D), lambda g: (0, 0)),
            pl.BlockSpec((1, D), lambda g: (0, 0)),
        ],
        out_specs=pl.BlockSpec((_BR, D), lambda g: (g, 0)),
        out_shape=jax.ShapeDtypeStruct((N, D), jnp.float32),
    )(x, a, a, a, cnt, cnt, cnt, w, rw, b)


def _tc_pq_body(x_ref, wa_ref, wb_ref, b_ref, p_ref, q_ref):
    x = x_ref[...]
    p_ref[...] = jnp.dot(x, wa_ref[...], preferred_element_type=jnp.float32) + b_ref[...]
    q_ref[...] = jnp.dot(x, wb_ref[...], preferred_element_type=jnp.float32)


def _tc_pq(x, wa, wb, b):
    return pl.pallas_call(
        _tc_pq_body,
        grid=(N // _BR,),
        in_specs=[
            pl.BlockSpec((_BR, D), lambda g: (g, 0)),
            pl.BlockSpec((D, D), lambda g: (0, 0)),
            pl.BlockSpec((D, D), lambda g: (0, 0)),
            pl.BlockSpec((1, D), lambda g: (0, 0)),
        ],
        out_specs=[pl.BlockSpec((_BR, D), lambda g: (g, 0)),
                   pl.BlockSpec((_BR, D), lambda g: (g, 0))],
        out_shape=[jax.ShapeDtypeStruct((N, D), jnp.float32),
                   jax.ShapeDtypeStruct((N, D), jnp.float32)],
    )(x, wa, wb, b)


_BE = 512


def _tc_edge_body(gp_ref, gq_ref, w_ref, b_ref, o_ref):
    eh = jnp.maximum(gp_ref[...] + gq_ref[...], 0.0)
    o_ref[...] = jnp.dot(eh, w_ref[...], preferred_element_type=jnp.float32) + b_ref[...]


def _tc_edge(gp, gq, w, b):
    return pl.pallas_call(
        _tc_edge_body,
        grid=(E // _BE,),
        in_specs=[
            pl.BlockSpec((_BE, D), lambda g: (g, 0)),
            pl.BlockSpec((_BE, D), lambda g: (g, 0)),
            pl.BlockSpec((D, 8), lambda g: (0, 0)),
            pl.BlockSpec((1, 8), lambda g: (0, 0)),
        ],
        out_specs=pl.BlockSpec((_BE, 8), lambda g: (g, 0)),
        out_shape=jax.ShapeDtypeStruct((E, 8), jnp.float32),
    )(gp, gq, w, b)


def _tc_node_body(x_ref, w1_ref, b1_ref, w2_ref, b2_ref, o_ref):
    hdd = jnp.dot(x_ref[...], w1_ref[...], preferred_element_type=jnp.float32)
    hdd = jnp.maximum(hdd + b1_ref[...], 0.0)
    o_ref[...] = jnp.dot(hdd, w2_ref[...], preferred_element_type=jnp.float32) + b2_ref[...]


def _tc_node(x, w1, b1, w2, b2):
    return pl.pallas_call(
        _tc_node_body,
        grid=(N // _BR,),
        in_specs=[
            pl.BlockSpec((_BR, D), lambda g: (g, 0)),
            pl.BlockSpec((D, 64), lambda g: (0, 0)),
            pl.BlockSpec((1, 64), lambda g: (0, 0)),
            pl.BlockSpec((64, 8), lambda g: (0, 0)),
            pl.BlockSpec((1, 8), lambda g: (0, 0)),
        ],
        out_specs=pl.BlockSpec((_BR, 8), lambda g: (g, 0)),
        out_shape=jax.ShapeDtypeStruct((N, 8), jnp.float32),
    )(x, w1, b1, w2, b2)


def kernel(x, edge_index, edge_type, W1, R1, b1, W2, R2, b2, W3, R3, b3,
           We1, be1, We2, be2, Wn1, bn1, Wn2, bn2):
    src = edge_index[0]
    dst = edge_index[1]
    zrows = jnp.zeros((ASLAB, D), jnp.float32)
    ones_h = jnp.ones((CH, D), jnp.float32)

    keys, cnt = _sc_prep(dst, edge_type, ones_h, zrows)

    xcur = x
    for (W, Rw, b, relu) in ((W1, R1, b1, True), (W2, R2, b2, True),
                             (W3, R3, b3, False)):
        a = _sc_agg(xcur, src, keys, zrows)
        xcur = _tc_combine(xcur, a, cnt, W, Rw, b.reshape(1, D), relu)

    p, q = _tc_pq(xcur, We1[:D], We1[D:], be1.reshape(1, D))
    gp, gq = _sc_edge_gather(p, q, src, dst)

    we2p = jnp.zeros((D, 8), jnp.float32).at[:, :3].set(We2)
    be2p = jnp.zeros((1, 8), jnp.float32).at[0, :3].set(be2)
    edge_out = _tc_edge(gp, gq, we2p, be2p)[:, :3]

    wn2p = jnp.zeros((64, 8), jnp.float32).at[:, :2].set(Wn2)
    bn2p = jnp.zeros((1, 8), jnp.float32).at[0, :2].set(bn2)
    node_out = _tc_node(xcur, Wn1, bn1.reshape(1, 64), wn2p, bn2p)[:, :2]

    return edge_out, node_out


# speedup vs baseline: 3.8731x; 1.0806x over previous
import functools

import jax
import jax.numpy as jnp
from jax import lax
from jax.experimental import pallas as pl
from jax.experimental.pallas import tpu as pltpu
from jax.experimental.pallas import tpu_sc as plsc

N = 10000
E = 320000
D = 128
NR = 3
HALF = 5000
RPAD = 5120
GSLOTS = NR * 2 * RPAD
AROWS = 15008
ADUMMY = 15000
ASLAB = 944
CH = 80
CHA = 40
NT = 16
CAP = 10640
CAPP = CAP + 16

_mesh = plsc.VectorSubcoreMesh(core_axis_name="c", subcore_axis_name="s")


@functools.partial(
    pl.kernel,
    out_type=(jax.ShapeDtypeStruct((2 * E,), jnp.int32),
              jax.ShapeDtypeStruct((2, AROWS, D), jnp.float32)),
    mesh=_mesh,
    scratch_types=[
        pltpu.VMEM((CH,), jnp.int32),
        pltpu.VMEM((CH,), jnp.int32),
        pltpu.VMEM((CH,), jnp.int32),
        pltpu.VMEM((CH,), jnp.int32),
        pltpu.VMEM((CH, D), jnp.float32),
        pltpu.VMEM_SHARED((AROWS, D), jnp.float32),
        pltpu.SemaphoreType.DMA,
        pltpu.SemaphoreType.DMA,
    ],
)
def _sc_prep(dsts, et, ones_h, zrows, keys, cnt, dstb, etb, kb1, kb2, onesb,
             acc, as1, as2):
    h = lax.axis_index("c")
    s = lax.axis_index("s")

    @pl.when(s < NT - 1)
    def _():
        pltpu.sync_copy(zrows, acc.at[pl.ds(s * ASLAB, ASLAB)])

    @pl.when(s == NT - 1)
    def _():
        pltpu.sync_copy(zrows.at[pl.ds(0, AROWS - (NT - 1) * ASLAB)],
                        acc.at[pl.ds(s * ASLAB, AROWS - (NT - 1) * ASLAB)])

    pltpu.sync_copy(ones_h, onesb)
    plsc.subcore_barrier()
    per_tile = E // NT
    base = s * per_tile
    nch = per_tile // CH
    lo = h * HALF

    def compute(c, kb):
        off = base + c * CH
        pltpu.sync_copy(dsts.at[pl.ds(off, CH)], dstb)
        pltpu.sync_copy(et.at[pl.ds(off, CH)], etb)
        for j in range(CH // 16):
            sl = pl.ds(j * 16, 16)
            d = dstb[sl]
            t = etb[sl]
            mine = (d >= lo) & (d < lo + HALF)
            kb[sl] = jnp.where(mine, t * HALF + d - lo, ADUMMY)
        pltpu.sync_copy(kb, keys.at[pl.ds(h * E + off, CH)])

    def swait(sem):
        pltpu.make_async_copy(onesb, acc.at[pl.ds(0, CH)], sem).wait()

    compute(0, kb1)
    pltpu.async_copy(onesb, acc.at[kb1], as1, add=True)

    def body(k, carry):
        compute(2 * k + 1, kb2)
        pltpu.async_copy(onesb, acc.at[kb2], as2, add=True)
        swait(as1)
        compute(2 * k + 2, kb1)
        pltpu.async_copy(onesb, acc.at[kb1], as1, add=True)
        swait(as2)
        return carry

    lax.fori_loop(0, (nch - 2) // 2, body, 0)
    compute(nch - 1, kb2)
    pltpu.async_copy(onesb, acc.at[kb2], as2, add=True)
    swait(as1)
    swait(as2)
    plsc.subcore_barrier()

    @pl.when(s < NT - 1)
    def _():
        pltpu.sync_copy(acc.at[pl.ds(s * ASLAB, ASLAB)],
                        cnt.at[h, pl.ds(s * ASLAB, ASLAB)])

    @pl.when(s == NT - 1)
    def _():
        tail = AROWS - (NT - 1) * ASLAB
        pltpu.sync_copy(acc.at[pl.ds(s * ASLAB, tail)],
                        cnt.at[h, pl.ds(s * ASLAB, tail)])


@functools.partial(
    pl.kernel,
    out_type=jax.ShapeDtypeStruct((2, AROWS, D), jnp.float32),
    mesh=_mesh,
    scratch_types=[
        pltpu.VMEM((CHA,), jnp.int32),
        pltpu.VMEM((CHA,), jnp.int32),
        pltpu.VMEM((CHA,), jnp.int32),
        pltpu.VMEM((CHA,), jnp.int32),
        pltpu.VMEM((CHA, D), jnp.float32),
        pltpu.VMEM((CHA, D), jnp.float32),
        pltpu.VMEM_SHARED((AROWS, D), jnp.float32),
        pltpu.SemaphoreType.DMA,
        pltpu.SemaphoreType.DMA,
    ],
)
def _sc_agg(x, srcs, keys, zrows, a_out, srcb1, srcb2, keyb1, keyb2, rows1,
            rows2, acc, sem1, sem2):
    h = lax.axis_index("c")
    s = lax.axis_index("s")

    @pl.when(s < NT - 1)
    def _():
        pltpu.sync_copy(zrows, acc.at[pl.ds(s * ASLAB, ASLAB)])

    @pl.when(s == NT - 1)
    def _():
        pltpu.sync_copy(zrows.at[pl.ds(0, AROWS - (NT - 1) * ASLAB)],
                        acc.at[pl.ds(s * ASLAB, AROWS - (NT - 1) * ASLAB)])

    plsc.subcore_barrier()
    base = s * (E // NT)
    nch = (E // NT) // CHA

    def load(off, srcb, keyb, rows, sem):
        pltpu.sync_copy(srcs.at[pl.ds(off, CHA)], srcb)
        pltpu.sync_copy(keys.at[pl.ds(h * E + off, CHA)], keyb)
        return pltpu.async_copy(x.at[srcb], rows, sem)

    load(base, srcb1, keyb1, rows1, sem1)

    def body(k, carry):
        offb = base + (2 * k + 1) * CHA
        offa = base + (2 * k + 2) * CHA
        load(offb, srcb2, keyb2, rows2, sem2)
        pltpu.make_async_copy(x.at[srcb1], rows1, sem1).wait()
        pltpu.sync_copy(rows1, acc.at[keyb1], add=True)
        load(offa, srcb1, keyb1, rows1, sem1)
        pltpu.make_async_copy(x.at[srcb2], rows2, sem2).wait()
        pltpu.sync_copy(rows2, acc.at[keyb2], add=True)
        return carry

    lax.fori_loop(0, (nch - 2) // 2, body, 0)
    load(base + (nch - 1) * CHA, srcb2, keyb2, rows2, sem2)
    pltpu.make_async_copy(x.at[srcb1], rows1, sem1).wait()
    pltpu.sync_copy(rows1, acc.at[keyb1], add=True)
    pltpu.make_async_copy(x.at[srcb2], rows2, sem2).wait()
    pltpu.sync_copy(rows2, acc.at[keyb2], add=True)
    plsc.subcore_barrier()

    @pl.when(s < NT - 1)
    def _():
        pltpu.sync_copy(acc.at[pl.ds(s * ASLAB, ASLAB)],
                        a_out.at[h, pl.ds(s * ASLAB, ASLAB)])

    @pl.when(s == NT - 1)
    def _():
        tail = AROWS - (NT - 1) * ASLAB
        pltpu.sync_copy(acc.at[pl.ds(s * ASLAB, tail)],
                        a_out.at[h, pl.ds(s * ASLAB, tail)])


@functools.partial(
    pl.kernel,
    out_type=(jax.ShapeDtypeStruct((E, D), jnp.float32),
              jax.ShapeDtypeStruct((E, D), jnp.float32)),
    mesh=_mesh,
    scratch_types=[
        pltpu.VMEM((CH,), jnp.int32),
        pltpu.VMEM((CH,), jnp.int32),
        pltpu.VMEM((CH,), jnp.int32),
        pltpu.VMEM((CH,), jnp.int32),
        pltpu.VMEM((CH, D), jnp.float32),
        pltpu.VMEM((CH, D), jnp.float32),
        pltpu.VMEM((CH, D), jnp.float32),
        pltpu.VMEM((CH, D), jnp.float32),
        pltpu.SemaphoreType.DMA,
        pltpu.SemaphoreType.DMA,
        pltpu.SemaphoreType.DMA,
        pltpu.SemaphoreType.DMA,
    ],
)
def _sc_edge_gather(p, q, srcs, dsts, gp, gq, srcb1, dstb1, srcb2, dstb2,
                    rp1, rq1, rp2, rq2, sp1, sq1, sp2, sq2):
    h = lax.axis_index("c")
    s = lax.axis_index("s")
    wid = h * NT + s
    per_w = E // (2 * NT)
    base = wid * per_w
    nch = per_w // CH

    def load(c, srcb, dstb, rp, rq, sp, sq):
        off = base + c * CH
        pltpu.sync_copy(srcs.at[pl.ds(off, CH)], srcb)
        pltpu.sync_copy(dsts.at[pl.ds(off, CH)], dstb)
        pltpu.async_copy(p.at[srcb], rp, sp)
        pltpu.async_copy(q.at[dstb], rq, sq)

    def drain_write(c, srcb, dstb, rp, rq, sp, sq):
        off = base + c * CH
        pltpu.make_async_copy(p.at[srcb], rp, sp).wait()
        pltpu.make_async_copy(q.at[dstb], rq, sq).wait()
        pltpu.sync_copy(rp, gp.at[pl.ds(off, CH)])
        pltpu.sync_copy(rq, gq.at[pl.ds(off, CH)])

    load(0, srcb1, dstb1, rp1, rq1, sp1, sq1)

    def body(k, carry):
        load(2 * k + 1, srcb2, dstb2, rp2, rq2, sp2, sq2)
        drain_write(2 * k, srcb1, dstb1, rp1, rq1, sp1, sq1)
        load(2 * k + 2, srcb1, dstb1, rp1, rq1, sp1, sq1)
        drain_write(2 * k + 1, srcb2, dstb2, rp2, rq2, sp2, sq2)
        return carry

    lax.fori_loop(0, (nch - 1) // 2, body, 0)
    drain_write(nch - 1, srcb1, dstb1, rp1, rq1, sp1, sq1)


_BR = 200


def _tc_combine_body(relu, x_ref, a0_ref, a1_ref, a2_ref, c0_ref, c1_ref,
                     c2_ref, w_ref, rw_ref, b_ref, o_ref):
    acc = jnp.dot(x_ref[...], rw_ref[...], preferred_element_type=jnp.float32)
    acc = acc + b_ref[...]
    for r, (ar, cr) in enumerate(zip((a0_ref, a1_ref, a2_ref),
                                     (c0_ref, c1_ref, c2_ref))):
        c = cr[0][:, 0:1]
        a = ar[0] / jnp.maximum(c, 1.0)
        acc = acc + jnp.dot(a, w_ref[r], preferred_element_type=jnp.float32)
    o_ref[...] = jnp.maximum(acc, 0.0) if relu else acc


def _tc_combine(x, a, cnt, w, rw, b, relu):
    nb = HALF // _BR
    a_spec = lambda r: pl.BlockSpec(
        (1, _BR, D), lambda g, r=r: (g // nb, r * nb + (g % nb), 0))
    return pl.pallas_call(
        functools.partial(_tc_combine_body, relu),
        grid=(2 * nb,),
        in_specs=[
            pl.BlockSpec((_BR, D), lambda g: (g, 0)),
            a_spec(0), a_spec(1), a_spec(2),
            a_spec(0), a_spec(1), a_spec(2),
            pl.BlockSpec((NR, D, D), lambda g: (0, 0, 0)),
            pl.BlockSpec((D, D), lambda g: (0, 0)),
            pl.BlockSpec((1, D), lambda g: (0, 0)),
        ],
        out_specs=pl.BlockSpec((_BR, D), lambda g: (g, 0)),
        out_shape=jax.ShapeDtypeStruct((N, D), jnp.float32),
    )(x, a, a, a, cnt, cnt, cnt, w, rw, b)


def _tc_pq_body(x_ref, wa_ref, wb_ref, b_ref, p_ref, q_ref):
    x = x_ref[...]
    p_ref[...] = jnp.dot(x, wa_ref[...], preferred_element_type=jnp.float32) + b_ref[...]
    q_ref[...] = jnp.dot(x, wb_ref[...], preferred_element_type=jnp.float32)


def _tc_pq(x, wa, wb, b):
    return pl.pallas_call(
        _tc_pq_body,
        grid=(N // _BR,),
        in_specs=[
            pl.BlockSpec((_BR, D), lambda g: (g, 0)),
            pl.BlockSpec((D, D), lambda g: (0, 0)),
            pl.BlockSpec((D, D), lambda g: (0, 0)),
            pl.BlockSpec((1, D), lambda g: (0, 0)),
        ],
        out_specs=[pl.BlockSpec((_BR, D), lambda g: (g, 0)),
                   pl.BlockSpec((_BR, D), lambda g: (g, 0))],
        out_shape=[jax.ShapeDtypeStruct((N, D), jnp.float32),
                   jax.ShapeDtypeStruct((N, D), jnp.float32)],
    )(x, wa, wb, b)


_BE = 512


def _tc_edge_body(gp_ref, gq_ref, w_ref, b_ref, o_ref):
    eh = jnp.maximum(gp_ref[...] + gq_ref[...], 0.0)
    o_ref[...] = jnp.dot(eh, w_ref[...], preferred_element_type=jnp.float32) + b_ref[...]


def _tc_edge(gp, gq, w, b):
    return pl.pallas_call(
        _tc_edge_body,
        grid=(E // _BE,),
        in_specs=[
            pl.BlockSpec((_BE, D), lambda g: (g, 0)),
            pl.BlockSpec((_BE, D), lambda g: (g, 0)),
            pl.BlockSpec((D, 8), lambda g: (0, 0)),
            pl.BlockSpec((1, 8), lambda g: (0, 0)),
        ],
        out_specs=pl.BlockSpec((_BE, 8), lambda g: (g, 0)),
        out_shape=jax.ShapeDtypeStruct((E, 8), jnp.float32),
    )(gp, gq, w, b)


def _tc_node_body(x_ref, w1_ref, b1_ref, w2_ref, b2_ref, o_ref):
    hdd = jnp.dot(x_ref[...], w1_ref[...], preferred_element_type=jnp.float32)
    hdd = jnp.maximum(hdd + b1_ref[...], 0.0)
    o_ref[...] = jnp.dot(hdd, w2_ref[...], preferred_element_type=jnp.float32) + b2_ref[...]


def _tc_node(x, w1, b1, w2, b2):
    return pl.pallas_call(
        _tc_node_body,
        grid=(N // _BR,),
        in_specs=[
            pl.BlockSpec((_BR, D), lambda g: (g, 0)),
            pl.BlockSpec((D, 64), lambda g: (0, 0)),
            pl.BlockSpec((1, 64), lambda g: (0, 0)),
            pl.BlockSpec((64, 8), lambda g: (0, 0)),
            pl.BlockSpec((1, 8), lambda g: (0, 0)),
        ],
        out_specs=pl.BlockSpec((_BR, 8), lambda g: (g, 0)),
        out_shape=jax.ShapeDtypeStruct((N, 8), jnp.float32),
    )(x, w1, b1, w2, b2)


def kernel(x, edge_index, edge_type, W1, R1, b1, W2, R2, b2, W3, R3, b3,
           We1, be1, We2, be2, Wn1, bn1, Wn2, bn2):
    src = edge_index[0]
    dst = edge_index[1]
    zrows = jnp.zeros((ASLAB, D), jnp.float32)
    ones_h = jnp.ones((CH, D), jnp.float32)

    keys, cnt = _sc_prep(dst, edge_type, ones_h, zrows)

    xcur = x
    for (W, Rw, b, relu) in ((W1, R1, b1, True), (W2, R2, b2, True),
                             (W3, R3, b3, False)):
        a = _sc_agg(xcur, src, keys, zrows)
        xcur = _tc_combine(xcur, a, cnt, W, Rw, b.reshape(1, D), relu)

    p, q = _tc_pq(xcur, We1[:D], We1[D:], be1.reshape(1, D))
    gp, gq = _sc_edge_gather(p, q, src, dst)

    we2p = jnp.zeros((D, 8), jnp.float32).at[:, :3].set(We2)
    be2p = jnp.zeros((1, 8), jnp.float32).at[0, :3].set(be2)
    edge_out = _tc_edge(gp, gq, we2p, be2p)[:, :3]

    wn2p = jnp.zeros((64, 8), jnp.float32).at[:, :2].set(Wn2)
    bn2p = jnp.zeros((1, 8), jnp.float32).at[0, :2].set(bn2)
    node_out = _tc_node(xcur, Wn1, bn1.reshape(1, 64), wn2p, bn2p)[:, :2]

    return edge_out, node_out


# speedup vs baseline: 3.9421x; 1.0178x over previous
import functools

import jax
import jax.numpy as jnp
from jax import lax
from jax.experimental import pallas as pl
from jax.experimental.pallas import tpu as pltpu
from jax.experimental.pallas import tpu_sc as plsc

N = 10000
E = 320000
D = 128
NR = 3
HALF = 5000
RPAD = 5120
GSLOTS = NR * 2 * RPAD
AROWS = 15008
ADUMMY = 15000
ASLAB = 944
CH = 80
CHA = 40
NT = 16
CAP = 10640
CAPP = CAP + 16

_mesh = plsc.VectorSubcoreMesh(core_axis_name="c", subcore_axis_name="s")


@functools.partial(
    pl.kernel,
    out_type=(jax.ShapeDtypeStruct((2 * E,), jnp.int32),
              jax.ShapeDtypeStruct((2, AROWS, D), jnp.float32)),
    mesh=_mesh,
    scratch_types=[
        pltpu.VMEM((CH,), jnp.int32),
        pltpu.VMEM((CH,), jnp.int32),
        pltpu.VMEM((CH,), jnp.int32),
        pltpu.VMEM((CH,), jnp.int32),
        pltpu.VMEM((CH, D), jnp.float32),
        pltpu.VMEM_SHARED((AROWS, D), jnp.float32),
        pltpu.SemaphoreType.DMA,
        pltpu.SemaphoreType.DMA,
    ],
)
def _sc_prep(dsts, et, ones_h, zrows, keys, cnt, dstb, etb, kb1, kb2, onesb,
             acc, as1, as2):
    h = lax.axis_index("c")
    s = lax.axis_index("s")

    @pl.when(s < NT - 1)
    def _():
        pltpu.sync_copy(zrows, acc.at[pl.ds(s * ASLAB, ASLAB)])

    @pl.when(s == NT - 1)
    def _():
        pltpu.sync_copy(zrows.at[pl.ds(0, AROWS - (NT - 1) * ASLAB)],
                        acc.at[pl.ds(s * ASLAB, AROWS - (NT - 1) * ASLAB)])

    pltpu.sync_copy(ones_h, onesb)
    plsc.subcore_barrier()
    per_tile = E // NT
    base = s * per_tile
    nch = per_tile // CH
    lo = h * HALF

    def compute(c, kb):
        off = base + c * CH
        pltpu.sync_copy(dsts.at[pl.ds(off, CH)], dstb)
        pltpu.sync_copy(et.at[pl.ds(off, CH)], etb)
        for j in range(CH // 16):
            sl = pl.ds(j * 16, 16)
            d = dstb[sl]
            t = etb[sl]
            mine = (d >= lo) & (d < lo + HALF)
            kb[sl] = jnp.where(mine, t * HALF + d - lo, ADUMMY)
        pltpu.sync_copy(kb, keys.at[pl.ds(h * E + off, CH)])

    def swait(sem):
        pltpu.make_async_copy(onesb, acc.at[pl.ds(0, CH)], sem).wait()

    compute(0, kb1)
    pltpu.async_copy(onesb, acc.at[kb1], as1, add=True)

    def body(k, carry):
        compute(2 * k + 1, kb2)
        pltpu.async_copy(onesb, acc.at[kb2], as2, add=True)
        swait(as1)
        compute(2 * k + 2, kb1)
        pltpu.async_copy(onesb, acc.at[kb1], as1, add=True)
        swait(as2)
        return carry

    lax.fori_loop(0, (nch - 2) // 2, body, 0)
    compute(nch - 1, kb2)
    pltpu.async_copy(onesb, acc.at[kb2], as2, add=True)
    swait(as1)
    swait(as2)
    plsc.subcore_barrier()

    @pl.when(s < NT - 1)
    def _():
        pltpu.sync_copy(acc.at[pl.ds(s * ASLAB, ASLAB)],
                        cnt.at[h, pl.ds(s * ASLAB, ASLAB)])

    @pl.when(s == NT - 1)
    def _():
        tail = AROWS - (NT - 1) * ASLAB
        pltpu.sync_copy(acc.at[pl.ds(s * ASLAB, tail)],
                        cnt.at[h, pl.ds(s * ASLAB, tail)])


@functools.partial(
    pl.kernel,
    out_type=jax.ShapeDtypeStruct((2, AROWS, D), jnp.float32),
    mesh=_mesh,
    scratch_types=[
        pltpu.VMEM((CHA,), jnp.int32),
        pltpu.VMEM((CHA,), jnp.int32),
        pltpu.VMEM((CHA,), jnp.int32),
        pltpu.VMEM((CHA,), jnp.int32),
        pltpu.VMEM((CHA, D), jnp.float32),
        pltpu.VMEM((CHA, D), jnp.float32),
        pltpu.VMEM_SHARED((AROWS, D), jnp.float32),
        pltpu.SemaphoreType.DMA,
        pltpu.SemaphoreType.DMA,
    ],
)
def _sc_agg(x, srcs, keys, zrows, a_out, srcb1, srcb2, keyb1, keyb2, rows1,
            rows2, acc, sem1, sem2):
    h = lax.axis_index("c")
    s = lax.axis_index("s")

    @pl.when(s < NT - 1)
    def _():
        pltpu.sync_copy(zrows, acc.at[pl.ds(s * ASLAB, ASLAB)])

    @pl.when(s == NT - 1)
    def _():
        pltpu.sync_copy(zrows.at[pl.ds(0, AROWS - (NT - 1) * ASLAB)],
                        acc.at[pl.ds(s * ASLAB, AROWS - (NT - 1) * ASLAB)])

    plsc.subcore_barrier()
    base = s * (E // NT)
    nch = (E // NT) // CHA

    def load(off, srcb, keyb, rows, sem):
        pltpu.sync_copy(srcs.at[pl.ds(off, CHA)], srcb)
        pltpu.sync_copy(keys.at[pl.ds(h * E + off, CHA)], keyb)
        return pltpu.async_copy(x.at[srcb], rows, sem)

    load(base, srcb1, keyb1, rows1, sem1)

    def body(k, carry):
        offb = base + (2 * k + 1) * CHA
        offa = base + (2 * k + 2) * CHA
        load(offb, srcb2, keyb2, rows2, sem2)
        pltpu.make_async_copy(x.at[srcb1], rows1, sem1).wait()
        pltpu.sync_copy(rows1, acc.at[keyb1], add=True)
        load(offa, srcb1, keyb1, rows1, sem1)
        pltpu.make_async_copy(x.at[srcb2], rows2, sem2).wait()
        pltpu.sync_copy(rows2, acc.at[keyb2], add=True)
        return carry

    lax.fori_loop(0, (nch - 2) // 2, body, 0)
    load(base + (nch - 1) * CHA, srcb2, keyb2, rows2, sem2)
    pltpu.make_async_copy(x.at[srcb1], rows1, sem1).wait()
    pltpu.sync_copy(rows1, acc.at[keyb1], add=True)
    pltpu.make_async_copy(x.at[srcb2], rows2, sem2).wait()
    pltpu.sync_copy(rows2, acc.at[keyb2], add=True)
    plsc.subcore_barrier()

    @pl.when(s < NT - 1)
    def _():
        pltpu.sync_copy(acc.at[pl.ds(s * ASLAB, ASLAB)],
                        a_out.at[h, pl.ds(s * ASLAB, ASLAB)])

    @pl.when(s == NT - 1)
    def _():
        tail = AROWS - (NT - 1) * ASLAB
        pltpu.sync_copy(acc.at[pl.ds(s * ASLAB, tail)],
                        a_out.at[h, pl.ds(s * ASLAB, tail)])


@functools.partial(
    pl.kernel,
    out_type=jax.ShapeDtypeStruct((E, D), jnp.float32),
    mesh=_mesh,
    scratch_types=[
        pltpu.VMEM((CH,), jnp.int32),
        pltpu.VMEM((CH,), jnp.int32),
        pltpu.VMEM((CH,), jnp.int32),
        pltpu.VMEM((CH,), jnp.int32),
        pltpu.VMEM((CH, D), jnp.float32),
        pltpu.VMEM((CH, D), jnp.float32),
        pltpu.VMEM((CH, D), jnp.float32),
        pltpu.VMEM((CH, D), jnp.float32),
        pltpu.SemaphoreType.DMA,
        pltpu.SemaphoreType.DMA,
        pltpu.SemaphoreType.DMA,
        pltpu.SemaphoreType.DMA,
    ],
)
def _sc_edge_gather(p, q, srcs, dsts, g, srcb1, dstb1, srcb2, dstb2,
                    rp1, rq1, rp2, rq2, sp1, sq1, sp2, sq2):
    h = lax.axis_index("c")
    s = lax.axis_index("s")
    wid = h * NT + s
    per_w = E // (2 * NT)
    base = wid * per_w
    nch = per_w // CH

    def load(c, srcb, dstb, rp, rq, sp, sq):
        off = base + c * CH
        pltpu.sync_copy(srcs.at[pl.ds(off, CH)], srcb)
        pltpu.sync_copy(dsts.at[pl.ds(off, CH)], dstb)
        pltpu.async_copy(p.at[srcb], rp, sp)
        pltpu.async_copy(q.at[dstb], rq, sq)

    def drain_write(c, srcb, dstb, rp, rq, sp, sq):
        off = base + c * CH
        pltpu.make_async_copy(p.at[srcb], rp, sp).wait()
        pltpu.make_async_copy(q.at[dstb], rq, sq).wait()

        def addrow(i, carry):
            for jj in range(D // 16):
                sl = pl.ds(jj * 16, 16)
                rp[i, sl] = rp[i, sl] + rq[i, sl]
            return carry

        lax.fori_loop(0, CH, addrow, 0)
        pltpu.sync_copy(rp, g.at[pl.ds(off, CH)])

    load(0, srcb1, dstb1, rp1, rq1, sp1, sq1)

    def body(k, carry):
        load(2 * k + 1, srcb2, dstb2, rp2, rq2, sp2, sq2)
        drain_write(2 * k, srcb1, dstb1, rp1, rq1, sp1, sq1)
        load(2 * k + 2, srcb1, dstb1, rp1, rq1, sp1, sq1)
        drain_write(2 * k + 1, srcb2, dstb2, rp2, rq2, sp2, sq2)
        return carry

    lax.fori_loop(0, (nch - 1) // 2, body, 0)
    drain_write(nch - 1, srcb1, dstb1, rp1, rq1, sp1, sq1)


_BR = 200


def _tc_combine_body(relu, x_ref, a0_ref, a1_ref, a2_ref, c0_ref, c1_ref,
                     c2_ref, w_ref, rw_ref, b_ref, o_ref):
    acc = jnp.dot(x_ref[...], rw_ref[...], preferred_element_type=jnp.float32)
    acc = acc + b_ref[...]
    for r, (ar, cr) in enumerate(zip((a0_ref, a1_ref, a2_ref),
                                     (c0_ref, c1_ref, c2_ref))):
        c = cr[0][:, 0:1]
        a = ar[0] / jnp.maximum(c, 1.0)
        acc = acc + jnp.dot(a, w_ref[r], preferred_element_type=jnp.float32)
    o_ref[...] = jnp.maximum(acc, 0.0) if relu else acc


def _tc_combine(x, a, cnt, w, rw, b, relu):
    nb = HALF // _BR
    a_spec = lambda r: pl.BlockSpec(
        (1, _BR, D), lambda g, r=r: (g // nb, r * nb + (g % nb), 0))
    return pl.pallas_call(
        functools.partial(_tc_combine_body, relu),
        grid=(2 * nb,),
        in_specs=[
            pl.BlockSpec((_BR, D), lambda g: (g, 0)),
            a_spec(0), a_spec(1), a_spec(2),
            a_spec(0), a_spec(1), a_spec(2),
            pl.BlockSpec((NR, D, D), lambda g: (0, 0, 0)),
            pl.BlockSpec((D, D), lambda g: (0, 0)),
            pl.BlockSpec((1, D), lambda g: (0, 0)),
        ],
        out_specs=pl.BlockSpec((_BR, D), lambda g: (g, 0)),
        out_shape=jax.ShapeDtypeStruct((N, D), jnp.float32),
    )(x, a, a, a, cnt, cnt, cnt, w, rw, b)


def _tc_pq_body(x_ref, wa_ref, wb_ref, b_ref, p_ref, q_ref):
    x = x_ref[...]
    p_ref[...] = jnp.dot(x, wa_ref[...], preferred_element_type=jnp.float32) + b_ref[...]
    q_ref[...] = jnp.dot(x, wb_ref[...], preferred_element_type=jnp.float32)


def _tc_pq(x, wa, wb, b):
    return pl.pallas_call(
        _tc_pq_body,
        grid=(N // _BR,),
        in_specs=[
            pl.BlockSpec((_BR, D), lambda g: (g, 0)),
            pl.BlockSpec((D, D), lambda g: (0, 0)),
            pl.BlockSpec((D, D), lambda g: (0, 0)),
            pl.BlockSpec((1, D), lambda g: (0, 0)),
        ],
        out_specs=[pl.BlockSpec((_BR, D), lambda g: (g, 0)),
                   pl.BlockSpec((_BR, D), lambda g: (g, 0))],
        out_shape=[jax.ShapeDtypeStruct((N, D), jnp.float32),
                   jax.ShapeDtypeStruct((N, D), jnp.float32)],
    )(x, wa, wb, b)


_BE = 512


def _tc_edge_body(g_ref, w_ref, b_ref, o_ref):
    eh = jnp.maximum(g_ref[...], 0.0)
    o_ref[...] = jnp.dot(eh, w_ref[...], preferred_element_type=jnp.float32) + b_ref[...]


def _tc_edge(g, w, b):
    return pl.pallas_call(
        _tc_edge_body,
        grid=(E // _BE,),
        in_specs=[
            pl.BlockSpec((_BE, D), lambda i: (i, 0)),
            pl.BlockSpec((D, 8), lambda i: (0, 0)),
            pl.BlockSpec((1, 8), lambda i: (0, 0)),
        ],
        out_specs=pl.BlockSpec((_BE, 8), lambda i: (i, 0)),
        out_shape=jax.ShapeDtypeStruct((E, 8), jnp.float32),
    )(g, w, b)


def _tc_node_body(x_ref, w1_ref, b1_ref, w2_ref, b2_ref, o_ref):
    hdd = jnp.dot(x_ref[...], w1_ref[...], preferred_element_type=jnp.float32)
    hdd = jnp.maximum(hdd + b1_ref[...], 0.0)
    o_ref[...] = jnp.dot(hdd, w2_ref[...], preferred_element_type=jnp.float32) + b2_ref[...]


def _tc_node(x, w1, b1, w2, b2):
    return pl.pallas_call(
        _tc_node_body,
        grid=(N // _BR,),
        in_specs=[
            pl.BlockSpec((_BR, D), lambda g: (g, 0)),
            pl.BlockSpec((D, 64), lambda g: (0, 0)),
            pl.BlockSpec((1, 64), lambda g: (0, 0)),
            pl.BlockSpec((64, 8), lambda g: (0, 0)),
            pl.BlockSpec((1, 8), lambda g: (0, 0)),
        ],
        out_specs=pl.BlockSpec((_BR, 8), lambda g: (g, 0)),
        out_shape=jax.ShapeDtypeStruct((N, 8), jnp.float32),
    )(x, w1, b1, w2, b2)


def kernel(x, edge_index, edge_type, W1, R1, b1, W2, R2, b2, W3, R3, b3,
           We1, be1, We2, be2, Wn1, bn1, Wn2, bn2):
    src = edge_index[0]
    dst = edge_index[1]
    zrows = jnp.zeros((ASLAB, D), jnp.float32)
    ones_h = jnp.ones((CH, D), jnp.float32)

    keys, cnt = _sc_prep(dst, edge_type, ones_h, zrows)

    xcur = x
    for (W, Rw, b, relu) in ((W1, R1, b1, True), (W2, R2, b2, True),
                             (W3, R3, b3, False)):
        a = _sc_agg(xcur, src, keys, zrows)
        xcur = _tc_combine(xcur, a, cnt, W, Rw, b.reshape(1, D), relu)

    p, q = _tc_pq(xcur, We1[:D], We1[D:], be1.reshape(1, D))
    g = _sc_edge_gather(p, q, src, dst)

    we2p = jnp.zeros((D, 8), jnp.float32).at[:, :3].set(We2)
    be2p = jnp.zeros((1, 8), jnp.float32).at[0, :3].set(be2)
    edge_out = _tc_edge(g, we2p, be2p)[:, :3]

    wn2p = jnp.zeros((64, 8), jnp.float32).at[:, :2].set(Wn2)
    bn2p = jnp.zeros((1, 8), jnp.float32).at[0, :2].set(bn2)
    node_out = _tc_node(xcur, Wn1, bn1.reshape(1, 64), wn2p, bn2p)[:, :2]

    return edge_out, node_out


# speedup vs baseline: 3.9780x; 1.0091x over previous
import functools

import jax
import jax.numpy as jnp
from jax import lax
from jax.experimental import pallas as pl
from jax.experimental.pallas import tpu as pltpu
from jax.experimental.pallas import tpu_sc as plsc

N = 10000
E = 320000
D = 128
NR = 3
HALF = 5000
RPAD = 5120
GSLOTS = NR * 2 * RPAD
AROWS = 15008
ADUMMY = 15000
ASLAB = 944
CH = 80
CHA = 40
NT = 16
CAP = 10640
CAPP = CAP + 16

_mesh = plsc.VectorSubcoreMesh(core_axis_name="c", subcore_axis_name="s")


@functools.partial(
    pl.kernel,
    out_type=(jax.ShapeDtypeStruct((2 * E,), jnp.int32),
              jax.ShapeDtypeStruct((2, AROWS, D), jnp.float32)),
    mesh=_mesh,
    scratch_types=[
        pltpu.VMEM((CH,), jnp.int32),
        pltpu.VMEM((CH,), jnp.int32),
        pltpu.VMEM((CH,), jnp.int32),
        pltpu.VMEM((CH,), jnp.int32),
        pltpu.VMEM((CH, D), jnp.float32),
        pltpu.VMEM_SHARED((AROWS, D), jnp.float32),
        pltpu.SemaphoreType.DMA,
        pltpu.SemaphoreType.DMA,
    ],
)
def _sc_prep(dsts, et, ones_h, zrows, keys, cnt, dstb, etb, kb1, kb2, onesb,
             acc, as1, as2):
    h = lax.axis_index("c")
    s = lax.axis_index("s")

    @pl.when(s < NT - 1)
    def _():
        pltpu.sync_copy(zrows, acc.at[pl.ds(s * ASLAB, ASLAB)])

    @pl.when(s == NT - 1)
    def _():
        pltpu.sync_copy(zrows.at[pl.ds(0, AROWS - (NT - 1) * ASLAB)],
                        acc.at[pl.ds(s * ASLAB, AROWS - (NT - 1) * ASLAB)])

    pltpu.sync_copy(ones_h, onesb)
    plsc.subcore_barrier()
    per_tile = E // NT
    base = s * per_tile
    nch = per_tile // CH
    lo = h * HALF

    def compute(c, kb):
        off = base + c * CH
        pltpu.sync_copy(dsts.at[pl.ds(off, CH)], dstb)
        pltpu.sync_copy(et.at[pl.ds(off, CH)], etb)
        for j in range(CH // 16):
            sl = pl.ds(j * 16, 16)
            d = dstb[sl]
            t = etb[sl]
            mine = (d >= lo) & (d < lo + HALF)
            kb[sl] = jnp.where(mine, t * HALF + d - lo, ADUMMY)
        pltpu.sync_copy(kb, keys.at[pl.ds(h * E + off, CH)])

    def swait(sem):
        pltpu.make_async_copy(onesb, acc.at[pl.ds(0, CH)], sem).wait()

    compute(0, kb1)
    pltpu.async_copy(onesb, acc.at[kb1], as1, add=True)

    def body(k, carry):
        compute(2 * k + 1, kb2)
        pltpu.async_copy(onesb, acc.at[kb2], as2, add=True)
        swait(as1)
        compute(2 * k + 2, kb1)
        pltpu.async_copy(onesb, acc.at[kb1], as1, add=True)
        swait(as2)
        return carry

    lax.fori_loop(0, (nch - 2) // 2, body, 0)
    compute(nch - 1, kb2)
    pltpu.async_copy(onesb, acc.at[kb2], as2, add=True)
    swait(as1)
    swait(as2)
    plsc.subcore_barrier()

    @pl.when(s < NT - 1)
    def _():
        pltpu.sync_copy(acc.at[pl.ds(s * ASLAB, ASLAB)],
                        cnt.at[h, pl.ds(s * ASLAB, ASLAB)])

    @pl.when(s == NT - 1)
    def _():
        tail = AROWS - (NT - 1) * ASLAB
        pltpu.sync_copy(acc.at[pl.ds(s * ASLAB, tail)],
                        cnt.at[h, pl.ds(s * ASLAB, tail)])


@functools.partial(
    pl.kernel,
    out_type=jax.ShapeDtypeStruct((2, AROWS, D), jnp.float32),
    mesh=_mesh,
    scratch_types=[
        pltpu.VMEM((CHA,), jnp.int32),
        pltpu.VMEM((CHA,), jnp.int32),
        pltpu.VMEM((CHA,), jnp.int32),
        pltpu.VMEM((CHA,), jnp.int32),
        pltpu.VMEM((CHA, D), jnp.float32),
        pltpu.VMEM((CHA, D), jnp.float32),
        pltpu.VMEM_SHARED((AROWS, D), jnp.float32),
        pltpu.SemaphoreType.DMA,
        pltpu.SemaphoreType.DMA,
    ],
)
def _sc_agg(x, srcs, keys, zrows, a_out, srcb1, srcb2, keyb1, keyb2, rows1,
            rows2, acc, sem1, sem2):
    h = lax.axis_index("c")
    s = lax.axis_index("s")

    @pl.when(s < NT - 1)
    def _():
        pltpu.sync_copy(zrows, acc.at[pl.ds(s * ASLAB, ASLAB)])

    @pl.when(s == NT - 1)
    def _():
        pltpu.sync_copy(zrows.at[pl.ds(0, AROWS - (NT - 1) * ASLAB)],
                        acc.at[pl.ds(s * ASLAB, AROWS - (NT - 1) * ASLAB)])

    plsc.subcore_barrier()
    base = s * (E // NT)
    nch = (E // NT) // CHA

    def load(off, srcb, keyb, rows, sem):
        pltpu.sync_copy(srcs.at[pl.ds(off, CHA)], srcb)
        pltpu.sync_copy(keys.at[pl.ds(h * E + off, CHA)], keyb)
        return pltpu.async_copy(x.at[srcb], rows, sem)

    load(base, srcb1, keyb1, rows1, sem1)

    def body(k, carry):
        offb = base + (2 * k + 1) * CHA
        offa = base + (2 * k + 2) * CHA
        load(offb, srcb2, keyb2, rows2, sem2)
        pltpu.make_async_copy(x.at[srcb1], rows1, sem1).wait()
        pltpu.sync_copy(rows1, acc.at[keyb1], add=True)
        load(offa, srcb1, keyb1, rows1, sem1)
        pltpu.make_async_copy(x.at[srcb2], rows2, sem2).wait()
        pltpu.sync_copy(rows2, acc.at[keyb2], add=True)
        return carry

    lax.fori_loop(0, (nch - 2) // 2, body, 0)
    load(base + (nch - 1) * CHA, srcb2, keyb2, rows2, sem2)
    pltpu.make_async_copy(x.at[srcb1], rows1, sem1).wait()
    pltpu.sync_copy(rows1, acc.at[keyb1], add=True)
    pltpu.make_async_copy(x.at[srcb2], rows2, sem2).wait()
    pltpu.sync_copy(rows2, acc.at[keyb2], add=True)
    plsc.subcore_barrier()

    @pl.when(s < NT - 1)
    def _():
        pltpu.sync_copy(acc.at[pl.ds(s * ASLAB, ASLAB)],
                        a_out.at[h, pl.ds(s * ASLAB, ASLAB)])

    @pl.when(s == NT - 1)
    def _():
        tail = AROWS - (NT - 1) * ASLAB
        pltpu.sync_copy(acc.at[pl.ds(s * ASLAB, tail)],
                        a_out.at[h, pl.ds(s * ASLAB, tail)])


@functools.partial(
    pl.kernel,
    out_type=jax.ShapeDtypeStruct((E, D), jnp.float32),
    mesh=_mesh,
    scratch_types=[
        pltpu.VMEM((CH,), jnp.int32),
        pltpu.VMEM((CH,), jnp.int32),
        pltpu.VMEM((CH,), jnp.int32),
        pltpu.VMEM((CH,), jnp.int32),
        pltpu.VMEM((CH, D), jnp.float32),
        pltpu.VMEM((CH, D), jnp.float32),
        pltpu.VMEM((CH, D), jnp.float32),
        pltpu.VMEM((CH, D), jnp.float32),
        pltpu.SemaphoreType.DMA,
        pltpu.SemaphoreType.DMA,
        pltpu.SemaphoreType.DMA,
        pltpu.SemaphoreType.DMA,
    ],
)
def _sc_edge_gather(p, q, srcs, dsts, g, srcb1, dstb1, srcb2, dstb2,
                    rp1, rq1, rp2, rq2, sp1, sq1, sp2, sq2):
    h = lax.axis_index("c")
    s = lax.axis_index("s")
    wid = h * NT + s
    per_w = E // (2 * NT)
    base = wid * per_w
    nch = per_w // CH

    def load(c, srcb, dstb, rp, rq, sp, sq):
        off = base + c * CH
        pltpu.sync_copy(srcs.at[pl.ds(off, CH)], srcb)
        pltpu.sync_copy(dsts.at[pl.ds(off, CH)], dstb)
        pltpu.async_copy(p.at[srcb], rp, sp)
        pltpu.async_copy(q.at[dstb], rq, sq)

    def drain_write(c, srcb, dstb, rp, rq, sp, sq):
        off = base + c * CH
        pltpu.make_async_copy(p.at[srcb], rp, sp).wait()
        pltpu.make_async_copy(q.at[dstb], rq, sq).wait()

        def addrow(i, carry):
            for jj in range(D // 16):
                sl = pl.ds(jj * 16, 16)
                rp[i, sl] = rp[i, sl] + rq[i, sl]
            return carry

        lax.fori_loop(0, CH, addrow, 0)
        pltpu.sync_copy(rp, g.at[pl.ds(off, CH)])

    load(0, srcb1, dstb1, rp1, rq1, sp1, sq1)

    def body(k, carry):
        load(2 * k + 1, srcb2, dstb2, rp2, rq2, sp2, sq2)
        drain_write(2 * k, srcb1, dstb1, rp1, rq1, sp1, sq1)
        load(2 * k + 2, srcb1, dstb1, rp1, rq1, sp1, sq1)
        drain_write(2 * k + 1, srcb2, dstb2, rp2, rq2, sp2, sq2)
        return carry

    lax.fori_loop(0, (nch - 1) // 2, body, 0)
    drain_write(nch - 1, srcb1, dstb1, rp1, rq1, sp1, sq1)


_BR = 200


def _tc_combine_body(relu, x_ref, a0_ref, a1_ref, a2_ref, c0_ref, c1_ref,
                     c2_ref, w_ref, rw_ref, b_ref, o_ref):
    acc = jnp.dot(x_ref[...], rw_ref[...], preferred_element_type=jnp.float32)
    acc = acc + b_ref[...]
    for r, (ar, cr) in enumerate(zip((a0_ref, a1_ref, a2_ref),
                                     (c0_ref, c1_ref, c2_ref))):
        c = cr[0][:, 0:1]
        a = ar[0] / jnp.maximum(c, 1.0)
        acc = acc + jnp.dot(a, w_ref[r], preferred_element_type=jnp.float32)
    o_ref[...] = jnp.maximum(acc, 0.0) if relu else acc


def _tc_combine(x, a, cnt, w, rw, b, relu):
    nb = HALF // _BR
    a_spec = lambda r: pl.BlockSpec(
        (1, _BR, D), lambda g, r=r: (g // nb, r * nb + (g % nb), 0))
    return pl.pallas_call(
        functools.partial(_tc_combine_body, relu),
        grid=(2 * nb,),
        in_specs=[
            pl.BlockSpec((_BR, D), lambda g: (g, 0)),
            a_spec(0), a_spec(1), a_spec(2),
            a_spec(0), a_spec(1), a_spec(2),
            pl.BlockSpec((NR, D, D), lambda g: (0, 0, 0)),
            pl.BlockSpec((D, D), lambda g: (0, 0)),
            pl.BlockSpec((1, D), lambda g: (0, 0)),
        ],
        out_specs=pl.BlockSpec((_BR, D), lambda g: (g, 0)),
        out_shape=jax.ShapeDtypeStruct((N, D), jnp.float32),
    )(x, a, a, a, cnt, cnt, cnt, w, rw, b)


def _tc_combine3_body(x_ref, a0_ref, a1_ref, a2_ref, c0_ref, c1_ref, c2_ref,
                      w_ref, rw_ref, b_ref, wa_ref, wb_ref, b1_ref,
                      o_ref, p_ref, q_ref):
    acc = jnp.dot(x_ref[...], rw_ref[...], preferred_element_type=jnp.float32)
    acc = acc + b_ref[...]
    for r, (ar, cr) in enumerate(zip((a0_ref, a1_ref, a2_ref),
                                     (c0_ref, c1_ref, c2_ref))):
        c = cr[0][:, 0:1]
        a = ar[0] / jnp.maximum(c, 1.0)
        acc = acc + jnp.dot(a, w_ref[r], preferred_element_type=jnp.float32)
    o_ref[...] = acc
    p_ref[...] = jnp.dot(acc, wa_ref[...], preferred_element_type=jnp.float32) + b1_ref[...]
    q_ref[...] = jnp.dot(acc, wb_ref[...], preferred_element_type=jnp.float32)


def _tc_combine3(x, a, cnt, w, rw, b, wa, wb, b1):
    nb = HALF // _BR
    a_spec = lambda r: pl.BlockSpec(
        (1, _BR, D), lambda g, r=r: (g // nb, r * nb + (g % nb), 0))
    row_spec = pl.BlockSpec((_BR, D), lambda g: (g, 0))
    full_spec = pl.BlockSpec((D, D), lambda g: (0, 0))
    return pl.pallas_call(
        _tc_combine3_body,
        grid=(2 * nb,),
        in_specs=[
            row_spec,
            a_spec(0), a_spec(1), a_spec(2),
            a_spec(0), a_spec(1), a_spec(2),
            pl.BlockSpec((NR, D, D), lambda g: (0, 0, 0)),
            full_spec,
            pl.BlockSpec((1, D), lambda g: (0, 0)),
            full_spec,
            full_spec,
            pl.BlockSpec((1, D), lambda g: (0, 0)),
        ],
        out_specs=[row_spec, row_spec, row_spec],
        out_shape=[jax.ShapeDtypeStruct((N, D), jnp.float32),
                   jax.ShapeDtypeStruct((N, D), jnp.float32),
                   jax.ShapeDtypeStruct((N, D), jnp.float32)],
    )(x, a, a, a, cnt, cnt, cnt, w, rw, b, wa, wb, b1)


_BE = 512


def _tc_edge_body(g_ref, w_ref, b_ref, o_ref):
    eh = jnp.maximum(g_ref[...], 0.0)
    o_ref[...] = jnp.dot(eh, w_ref[...], preferred_element_type=jnp.float32) + b_ref[...]


def _tc_edge(g, w, b):
    return pl.pallas_call(
        _tc_edge_body,
        grid=(E // _BE,),
        in_specs=[
            pl.BlockSpec((_BE, D), lambda i: (i, 0)),
            pl.BlockSpec((D, 8), lambda i: (0, 0)),
            pl.BlockSpec((1, 8), lambda i: (0, 0)),
        ],
        out_specs=pl.BlockSpec((_BE, 8), lambda i: (i, 0)),
        out_shape=jax.ShapeDtypeStruct((E, 8), jnp.float32),
    )(g, w, b)


def _tc_node_body(x_ref, w1_ref, b1_ref, w2_ref, b2_ref, o_ref):
    hdd = jnp.dot(x_ref[...], w1_ref[...], preferred_element_type=jnp.float32)
    hdd = jnp.maximum(hdd + b1_ref[...], 0.0)
    o_ref[...] = jnp.dot(hdd, w2_ref[...], preferred_element_type=jnp.float32) + b2_ref[...]


def _tc_node(x, w1, b1, w2, b2):
    return pl.pallas_call(
        _tc_node_body,
        grid=(N // _BR,),
        in_specs=[
            pl.BlockSpec((_BR, D), lambda g: (g, 0)),
            pl.BlockSpec((D, 64), lambda g: (0, 0)),
            pl.BlockSpec((1, 64), lambda g: (0, 0)),
            pl.BlockSpec((64, 8), lambda g: (0, 0)),
            pl.BlockSpec((1, 8), lambda g: (0, 0)),
        ],
        out_specs=pl.BlockSpec((_BR, 8), lambda g: (g, 0)),
        out_shape=jax.ShapeDtypeStruct((N, 8), jnp.float32),
    )(x, w1, b1, w2, b2)


def kernel(x, edge_index, edge_type, W1, R1, b1, W2, R2, b2, W3, R3, b3,
           We1, be1, We2, be2, Wn1, bn1, Wn2, bn2):
    src = edge_index[0]
    dst = edge_index[1]
    zrows = jnp.zeros((ASLAB, D), jnp.float32)
    ones_h = jnp.ones((CH, D), jnp.float32)

    keys, cnt = _sc_prep(dst, edge_type, ones_h, zrows)

    xcur = x
    for (W, Rw, b) in ((W1, R1, b1), (W2, R2, b2)):
        a = _sc_agg(xcur, src, keys, zrows)
        xcur = _tc_combine(xcur, a, cnt, W, Rw, b.reshape(1, D), True)

    a = _sc_agg(xcur, src, keys, zrows)
    xcur, p, q = _tc_combine3(xcur, a, cnt, W3, R3, b3.reshape(1, D),
                              We1[:D], We1[D:], be1.reshape(1, D))
    g = _sc_edge_gather(p, q, src, dst)

    we2p = jnp.zeros((D, 8), jnp.float32).at[:, :3].set(We2)
    be2p = jnp.zeros((1, 8), jnp.float32).at[0, :3].set(be2)
    edge_out = _tc_edge(g, we2p, be2p)[:, :3]

    wn2p = jnp.zeros((64, 8), jnp.float32).at[:, :2].set(Wn2)
    bn2p = jnp.zeros((1, 8), jnp.float32).at[0, :2].set(bn2)
    node_out = _tc_node(xcur, Wn1, bn1.reshape(1, 64), wn2p, bn2p)[:, :2]

    return edge_out, node_out
